# Initial kernel scaffold; baseline (speedup 1.0000x reference)
import jax
import jax.numpy as jnp
from jax.experimental import pallas as pl


def kernel(node_feats_1, edge_feats_1, node_feats_2, edge_feats_2, params, edge_index_1, edge_index_2):
    raise NotImplementedError("write your pallas kernel here")


# speedup vs baseline: 7.0291x; 7.0291x over previous
import functools

import jax
import jax.numpy as jnp
from jax import lax
from jax.experimental import pallas as pl
from jax.experimental.pallas import tpu as pltpu
from jax.experimental.pallas import tpu_sc as plsc

N = 10000
E = 320000
D = 128
DE = 16
G = 200
W = 256
WS = 16
HALF = W // 2

IDX_MINOR = 80
IDX_ROWS = E // IDX_MINOR
ROWS_PER_WORKER = IDX_ROWS // 32
ROWS_PER_TILE = IDX_ROWS // 16
NODE_CHUNK = 624
NODE_TAIL = N - 16 * NODE_CHUNK

def _leaky(x):
    return jnp.where(x >= 0, x, 0.01 * x)


def _elu(x):
    return jnp.where(x > 0, x, jnp.exp(jnp.minimum(x, 0.0)) - 1.0)


@functools.cache
def _mesh():
    return plsc.VectorSubcoreMesh(core_axis_name="c", subcore_axis_name="s")


@functools.cache
def _make_gather(width):
    @functools.partial(
        pl.kernel,
        mesh=_mesh(),
        out_type=jax.ShapeDtypeStruct((E, width), jnp.float32),
        scratch_types=[
            pltpu.VMEM((ROWS_PER_WORKER, IDX_MINOR), jnp.int32),
            pltpu.VMEM((IDX_MINOR, width), jnp.float32),
            pltpu.SemaphoreType.DMA,
        ],
    )
    def gather(table_hbm, idx_hbm, out_hbm, idx_v, rows_v, sem):
        wid = lax.axis_index("s") * 2 + lax.axis_index("c")
        base = wid * ROWS_PER_WORKER
        pltpu.sync_copy(idx_hbm.at[wid], idx_v)

        def body(i, carry):
            pltpu.async_copy(table_hbm.at[idx_v.at[i]], rows_v, sem).wait()
            pltpu.sync_copy(
                rows_v, out_hbm.at[pl.ds((base + i) * IDX_MINOR, IDX_MINOR)])
            return carry

        lax.fori_loop(0, ROWS_PER_WORKER, body, 0)

    return gather


def _gather_wide(table, idx3d):
    return _make_gather(W)(table, idx3d)


def _gather_scalar(table_n128, idx3d):
    return _make_gather(HALF)(table_n128, idx3d)


@functools.cache
def _make_scatter():
    @functools.partial(
        pl.kernel,
        mesh=_mesh(),
        out_type=jax.ShapeDtypeStruct((N, W), jnp.float32),
        scratch_types=[
            pltpu.VMEM((ROWS_PER_TILE, IDX_MINOR), jnp.int32),
            pltpu.VMEM((IDX_MINOR, HALF), jnp.float32),
            pltpu.VMEM_SHARED((N, HALF), jnp.float32),
        ],
    )
    def scatter(data_hbm, idx_hbm, zeros_hbm, out_hbm, idx_v, data_v, acc):
        c = lax.axis_index("c")
        s = lax.axis_index("s")
        pltpu.sync_copy(zeros_hbm,
                        acc.at[pl.ds(s * NODE_CHUNK, NODE_CHUNK)])

        @pl.when(s == 15)
        def _():
            pltpu.sync_copy(zeros_hbm.at[pl.ds(0, NODE_TAIL)],
                            acc.at[pl.ds(16 * NODE_CHUNK, NODE_TAIL)])

        plsc.subcore_barrier()
        pltpu.sync_copy(idx_hbm.at[s], idx_v)

        def body(i, carry):
            row = (s * ROWS_PER_TILE + i) * IDX_MINOR
            pltpu.sync_copy(
                data_hbm.at[pl.ds(row, IDX_MINOR), pl.ds(c * HALF, HALF)],
                data_v)
            pltpu.sync_copy(data_v, acc.at[idx_v.at[i]], add=True)
            return carry

        lax.fori_loop(0, ROWS_PER_TILE, body, 0)
        plsc.subcore_barrier()
        pltpu.sync_copy(
            acc.at[pl.ds(s * NODE_CHUNK, NODE_CHUNK)],
            out_hbm.at[pl.ds(s * NODE_CHUNK, NODE_CHUNK),
                       pl.ds(c * HALF, HALF)])

        @pl.when(s == 15)
        def _():
            pltpu.sync_copy(
                acc.at[pl.ds(16 * NODE_CHUNK, NODE_TAIL)],
                out_hbm.at[pl.ds(16 * NODE_CHUNK, NODE_TAIL),
                           pl.ds(c * HALF, HALF)])

    return scatter


def _scatter_add(data, idx2d, zeros_half):
    return _make_scatter()(data, idx2d, zeros_half)


NB = 2000
NGRID = N // NB
EB = 2000
EGRID = E // EB


def _dot(a, b):
    return jnp.dot(a, b, preferred_element_type=jnp.float32)


def _gru_math(x, h, wihT, bih, whhT, bhh):
    gi = _dot(x, wihT) + bih
    gh = _dot(h, whhT) + bhh
    g = gi.shape[-1] // 3
    r = jax.nn.sigmoid(gi[:, 0:g] + gh[:, 0:g])
    z = jax.nn.sigmoid(gi[:, g:2 * g] + gh[:, g:2 * g])
    n = jnp.tanh(gi[:, 2 * g:] + r * gh[:, 2 * g:])
    return (1.0 - z) * n + z * h


def _full(shape):
    nd = len(shape)
    return pl.BlockSpec(shape, lambda i: (0,) * nd)


def _prep_init_body(nf, wpnT, bpn, waT, u2, hv_o, ptab_o, sd_o):
    hv = _leaky(_dot(nf[:], wpnT[:]) + bpn[:])
    hv_o[:] = hv
    p = _dot(nf[:], waT[:])
    ptab_o[:] = jnp.concatenate(
        [p, jnp.zeros((p.shape[0], W - G), jnp.float32)], axis=1)
    sd = _dot(hv, u2[:])
    sd_o[:] = jnp.concatenate(
        [sd, jnp.zeros((sd.shape[0], HALF - 1), jnp.float32)], axis=1)


def _prep_init(nf, wpnT, bpn, waT, u2):
    return pl.pallas_call(
        _prep_init_body,
        grid=(NGRID,),
        in_specs=[
            pl.BlockSpec((NB, D), lambda i: (i, 0)),
            _full((D, G)), _full((1, G)), _full((D, G)), _full((G, 1)),
        ],
        out_specs=[
            pl.BlockSpec((NB, G), lambda i: (i, 0)),
            pl.BlockSpec((NB, W), lambda i: (i, 0)),
            pl.BlockSpec((NB, HALF), lambda i: (i, 0)),
        ],
        out_shape=[
            jax.ShapeDtypeStruct((N, G), jnp.float32),
            jax.ShapeDtypeStruct((N, W), jnp.float32),
            jax.ShapeDtypeStruct((N, HALF), jnp.float32),
        ],
    )(nf, wpnT, bpn, waT, u2)


def _edge_init_body(pg, ef, sdg, wbT, b1, v2, b2, wetT, bet, out_o):
    he1 = _leaky(pg[:, 0:G] + _dot(ef[:], wbT[:]) + b1[:])
    t = _dot(he1, v2[:])
    logit = _leaky(sdg[:, 0:1] + t + b2[:])
    w = jnp.exp(logit)
    msg = _dot(he1, wetT[:]) + bet[:]
    out_o[:] = jnp.concatenate(
        [w * msg, w, jnp.zeros((he1.shape[0], W - G - 1), jnp.float32)],
        axis=1)


def _edge_init(pg, ef, sdg, wbT, b1, v2, b2, wetT, bet):
    return pl.pallas_call(
        _edge_init_body,
        grid=(EGRID,),
        in_specs=[
            pl.BlockSpec((EB, W), lambda i: (i, 0)),
            pl.BlockSpec((EB, DE), lambda i: (i, 0)),
            pl.BlockSpec((EB, HALF), lambda i: (i, 0)),
            _full((DE, G)), _full((1, G)), _full((G, 1)), _full((1, 1)),
            _full((G, G)), _full((1, G)),
        ],
        out_specs=pl.BlockSpec((EB, W), lambda i: (i, 0)),
        out_shape=jax.ShapeDtypeStruct((E, W), jnp.float32),
    )(pg, ef, sdg, wbT, b1, v2, b2, wetT, bet)


def _edge_layer_body(projg, sdg, bl, out_o):
    logit = _leaky(sdg[:, 0:1] + projg[:, G:G + 1] + bl[:])
    w = jnp.exp(logit)
    out_o[:] = jnp.concatenate(
        [w * projg[:, 0:G], w,
         jnp.zeros((projg.shape[0], W - G - 1), jnp.float32)], axis=1)


def _edge_layer(projg, sdg, bl):
    return pl.pallas_call(
        _edge_layer_body,
        grid=(EGRID,),
        in_specs=[
            pl.BlockSpec((EB, W), lambda i: (i, 0)),
            pl.BlockSpec((EB, HALF), lambda i: (i, 0)),
            _full((1, 1)),
        ],
        out_specs=pl.BlockSpec((EB, W), lambda i: (i, 0)),
        out_shape=jax.ShapeDtypeStruct((E, W), jnp.float32),
    )(projg, sdg, bl)


def _update_init_body(S, hv, wihT, bih, whhT, bhh,
                      wpnlT, bpnl, ul, vvl, h_o, ptab_o, sd_o):
    ssum = S[:, G:G + 1]
    rinv = 1.0 / (ssum + 1e-16)
    ctx = _elu(S[:, 0:G] * rinv)
    h = jax.nn.relu(_gru_math(ctx, hv[:], wihT[:], bih[:], whhT[:], bhh[:]))
    h_o[:] = h
    proj = _dot(h, wpnlT[:]) + bpnl[:]
    ss = _dot(h, vvl[:])
    ptab_o[:] = jnp.concatenate(
        [proj, ss, jnp.zeros((proj.shape[0], W - G - 1), jnp.float32)], axis=1)
    sd = _dot(h, ul[:])
    sd_o[:] = jnp.concatenate(
        [sd, jnp.zeros((sd.shape[0], HALF - 1), jnp.float32)], axis=1)


def _update_init(S, hv, gru_p, wpnlT, bpnl, ul, vvl):
    return pl.pallas_call(
        _update_init_body,
        grid=(NGRID,),
        in_specs=[
            pl.BlockSpec((NB, W), lambda i: (i, 0)),
            pl.BlockSpec((NB, G), lambda i: (i, 0)),
            _full((G, 3 * G)), _full((1, 3 * G)),
            _full((G, 3 * G)), _full((1, 3 * G)),
            _full((G, G)), _full((1, G)), _full((G, 1)), _full((G, 1)),
        ],
        out_specs=[
            pl.BlockSpec((NB, G), lambda i: (i, 0)),
            pl.BlockSpec((NB, W), lambda i: (i, 0)),
            pl.BlockSpec((NB, HALF), lambda i: (i, 0)),
        ],
        out_shape=[
            jax.ShapeDtypeStruct((N, G), jnp.float32),
            jax.ShapeDtypeStruct((N, W), jnp.float32),
            jax.ShapeDtypeStruct((N, HALF), jnp.float32),
        ],
    )(S, hv, *gru_p, wpnlT, bpnl, ul, vvl)


def _update_layer_body(S, h, wihT, bih, whhT, bhh, wpnrT, bpnr, w2r,
                       h2_o, hvp_o, zb_o):
    ssum = S[:, G:G + 1]
    ctx = _elu(S[:, 0:G] / (ssum + 1e-16))
    h2 = jax.nn.relu(_gru_math(ctx, h[:], wihT[:], bih[:], whhT[:], bhh[:]))
    h2_o[:] = h2
    hvp_o[:] = _dot(h2, wpnrT[:]) + bpnr[:]
    zb_o[:] = _dot(h2, w2r[:])


def _update_layer(S, h, gru_p, wpnrT, bpnr, w2r):
    return pl.pallas_call(
        _update_layer_body,
        grid=(NGRID,),
        in_specs=[
            pl.BlockSpec((NB, W), lambda i: (i, 0)),
            pl.BlockSpec((NB, G), lambda i: (i, 0)),
            _full((G, 3 * G)), _full((1, 3 * G)),
            _full((G, 3 * G)), _full((1, 3 * G)),
            _full((G, G)), _full((1, G)), _full((G, 1)),
        ],
        out_specs=[
            pl.BlockSpec((NB, G), lambda i: (i, 0)),
            pl.BlockSpec((NB, G), lambda i: (i, 0)),
            pl.BlockSpec((NB, 1), lambda i: (i, 0)),
        ],
        out_shape=[
            jax.ShapeDtypeStruct((N, G), jnp.float32),
            jax.ShapeDtypeStruct((N, G), jnp.float32),
            jax.ShapeDtypeStruct((N, 1), jnp.float32),
        ],
    )(S, h, *gru_p, wpnrT, bpnr, w2r)


def _readout_body(h2, hvp, zbase, w1r, br, wihT, bih, whhT, bhh, g_o):
    g = jnp.sum(h2[:], axis=0, keepdims=True)
    for _ in range(2):
        gb = jax.nn.relu(g)
        z = _leaky(_dot(gb, w1r[:]) + zbase[:] + br[:])
        zm = jnp.max(z, axis=0, keepdims=True)
        ez = jnp.exp(z - zm)
        a = ez / jnp.sum(ez, axis=0, keepdims=True)
        g_repr = jnp.sum(a * hvp[:], axis=0, keepdims=True)
        g = jax.nn.relu(
            _gru_math(_elu(g_repr), g, wihT[:], bih[:], whhT[:], bhh[:]))
    g_o[:] = g


def _readout(h2, hvp, zbase, w1r, br, gru_p):
    return pl.pallas_call(
        _readout_body,
        out_shape=jax.ShapeDtypeStruct((1, G), jnp.float32),
    )(h2, hvp, zbase, w1r, br, *gru_p)


def _predict_body(g1, g2, wpT, bp, out_o):
    out_o[:] = _dot(g1[:] + g2[:], wpT[:]) + bp[:]


def _predict(g1, g2, wpT, bp):
    return pl.pallas_call(
        _predict_body,
        out_shape=jax.ShapeDtypeStruct((1, 1), jnp.float32),
    )(g1, g2, wpT, bp)


def _gru_pack(p):
    return (p['w_ih'].T, p['b_ih'].reshape(1, -1),
            p['w_hh'].T, p['b_hh'].reshape(1, -1))


def _branch(p_gnn, p_read, nf, ef, src3g, dst3g, dst3s, zeros_half):
    pc = p_gnn['init_context']
    hv_new, ptab, sdtab = _prep_init(
        nf,
        pc['project_node']['w'].T, pc['project_node']['b'].reshape(1, G),
        pc['project_edge1']['w'][:, :D].T,
        pc['project_edge2']['w'][0, :G].reshape(G, 1))
    pg = _gather_wide(ptab, src3g)
    sdg = _gather_scalar(sdtab, dst3g)
    m = _edge_init(
        pg, ef, sdg,
        pc['project_edge1']['w'][:, D:].T,
        pc['project_edge1']['b'].reshape(1, G),
        pc['project_edge2']['w'][0, G:].reshape(G, 1),
        pc['project_edge2']['b'].reshape(1, 1),
        pc['edge_transform']['w'].T,
        pc['edge_transform']['b'].reshape(1, G))
    S = _scatter_add(m, dst3s, zeros_half)

    lp = p_gnn['layers'][0]
    h, ptab2, sdtab2 = _update_init(
        S, hv_new,
        _gru_pack(pc['gru']),
        lp['project_node']['w'].T, lp['project_node']['b'].reshape(1, G),
        lp['project_edge']['w'][0, :G].reshape(G, 1),
        lp['project_edge']['w'][0, G:].reshape(G, 1))
    pg2 = _gather_wide(ptab2, src3g)
    sdg2 = _gather_scalar(sdtab2, dst3g)
    m2 = _edge_layer(pg2, sdg2, lp['project_edge']['b'].reshape(1, 1))
    S2 = _scatter_add(m2, dst3s, zeros_half)
    h2, hvp, zbase = _update_layer(
        S2, h, _gru_pack(lp['gru']),
        p_read['project_nodes']['w'].T,
        p_read['project_nodes']['b'].reshape(1, G),
        p_read['compute_logits']['w'][0, G:].reshape(G, 1))

    return _readout(
        h2, hvp, zbase,
        p_read['compute_logits']['w'][0, :G].reshape(G, 1),
        p_read['compute_logits']['b'].reshape(1, 1),
        _gru_pack(p_read['gru']))


@jax.jit
def kernel(node_feats_1, edge_feats_1, node_feats_2, edge_feats_2, params,
           edge_index_1, edge_index_2):
    zeros_half = jnp.zeros((NODE_CHUNK, HALF), jnp.float32)
    idx = []
    for ei in (edge_index_1, edge_index_2):
        src = ei[0].astype(jnp.int32)
        dst = ei[1].astype(jnp.int32)
        idx.append((src.reshape(32, ROWS_PER_WORKER, IDX_MINOR),
                    dst.reshape(32, ROWS_PER_WORKER, IDX_MINOR),
                    dst.reshape(16, ROWS_PER_TILE, IDX_MINOR)))
    g1 = _branch(params['gnn_1'], params['readout_1'], node_feats_1,
                 edge_feats_1, *idx[0], zeros_half)
    g2 = _branch(params['gnn_2'], params['readout_2'], node_feats_2,
                 edge_feats_2, *idx[1], zeros_half)
    return _predict(g1, g2, params['predict']['w'].T,
                    params['predict']['b'].reshape(1, 1))
